# baseline (speedup 1.0000x reference)
import functools

import jax
import jax.numpy as jnp
from jax import lax
from jax.experimental import pallas as pl
from jax.experimental.pallas import tpu as pltpu
from jax.experimental.pallas import tpu_sc as plsc

NB = 64
NF = 1280
NG = 512
NIDS = 65536
LANES = 16
NWORKERS = 32
BPW = NB // NWORKERS


NRG = 8
NQ = 4
CHUNK = 8192


def _sc_mask_body(aggr_hbm, mask_hbm, ids_v, mask_v):
    cid = lax.axis_index("c")
    sid = lax.axis_index("s")
    wid = sid * 2 + cid
    rg = wid // NQ
    q = wid % NQ
    zeros = jnp.zeros((LANES,), jnp.float32)
    ones = jnp.ones((LANES,), jnp.float32)

    for r in range(8):

        @plsc.parallel_loop(0, NF, step=LANES, unroll=8)
        def _zero(i):
            mask_v[r, pl.ds(i, LANES)] = zeros

    for cq in range(NIDS // NQ // CHUNK):
        col = q * (NIDS // NQ) + cq * CHUNK
        pltpu.sync_copy(aggr_hbm.at[pl.ds(rg * 8, 8), pl.ds(col, CHUNK)], ids_v)
        for r in range(8):
            row = jnp.full((LANES,), r, jnp.int32)

            @plsc.parallel_loop(0, CHUNK, step=LANES, unroll=16)
            def _scatter(i):
                idx = ids_v[r, pl.ds(i, LANES)]
                plsc.store_scatter(mask_v, [row, idx], ones)

    pltpu.sync_copy(mask_v, mask_hbm.at[q, pl.ds(rg * 8, 8)])


@jax.jit
def _sc_mask(aggr_info):
    mesh = plsc.VectorSubcoreMesh(core_axis_name="c", subcore_axis_name="s")
    return pl.kernel(
        _sc_mask_body,
        out_type=jax.ShapeDtypeStruct((NQ, NB, NF), jnp.float32),
        mesh=mesh,
        scratch_types=[
            pltpu.VMEM((8, CHUNK), jnp.int32),
            pltpu.VMEM((8, NF), jnp.float32),
        ],
        compiler_params=pltpu.CompilerParams(needs_layout_passes=False),
    )(aggr_info)


def _reduce_body(flow_ref, out_ref):
    for bb in range(8):
        s = flow_ref[bb, 0:128] + flow_ref[bb, 128:256]
        for half in (64, 32, 16, 8, 4, 2, 1):
            s = s[0:half] + s[half : 2 * half]
        out_ref[bb] = s[0] * jnp.float32(2.0 / NG)


@jax.jit
def _tc_reduce(flow4):
    return pl.pallas_call(
        _reduce_body,
        grid=(NB // 8,),
        in_specs=[pl.BlockSpec((8, NG // 2, 2, NF), lambda b: (b, 0, 0, 0))],
        out_specs=pl.BlockSpec((8, 2, NF), lambda b: (b, 0, 0)),
        out_shape=jax.ShapeDtypeStruct((NB, 2, NF), jnp.float32),
    )(flow4)


def _combine_body(avg_ref, prob_ref, mask_ref, out_ref):
    counts = (
        mask_ref[0] + mask_ref[1] + mask_ref[2] + mask_ref[3]
    )
    ind = jnp.where(counts > 0.0, jnp.float32(1.0), jnp.float32(0.0))
    d = (avg_ref[...] - prob_ref[...]) * ind[:, None, :]
    out_ref[0, 0] = jnp.sum(d * d) * jnp.float32(1.0 / (NB * NF * 2))


@jax.jit
def _tc_combine(avg_t, prob_t, mask):
    return pl.pallas_call(
        _combine_body,
        out_specs=pl.BlockSpec(memory_space=pltpu.SMEM),
        out_shape=jax.ShapeDtypeStruct((1, 1), jnp.float32),
    )(avg_t, prob_t, mask)


def kernel(flow, prob, aggr_info):
    flow4 = jnp.transpose(flow, (0, 2, 3, 4, 1)).reshape(NB, NG // 2, 2, NF)
    prob_t = jnp.transpose(prob, (0, 2, 1))
    mask = _sc_mask(aggr_info)
    avg_t = _tc_reduce(flow4)
    loss = _tc_combine(avg_t, prob_t, mask)[0, 0]
    avg10 = jnp.transpose(avg_t[0, :, 0:10])
    return (loss, avg10)


# speedup vs baseline: 1.0092x; 1.0092x over previous
import jax
import jax.numpy as jnp
from jax import lax
from jax.experimental import pallas as pl
from jax.experimental.pallas import tpu as pltpu
from jax.experimental.pallas import tpu_sc as plsc

NB = 64
NF = 1280
NG = 512
NIDS = 65536
LANES = 16


NQ = 4
CHUNK = 8192


def _sc_mask_body(aggr_hbm, mask_hbm, ids_v, mask_v):
    cid = lax.axis_index("c")
    sid = lax.axis_index("s")
    wid = sid * 2 + cid
    rg = wid // NQ
    q = wid % NQ
    zeros = jnp.zeros((LANES,), jnp.float32)
    ones = jnp.ones((LANES,), jnp.float32)

    for r in range(8):

        @plsc.parallel_loop(0, NF, step=LANES, unroll=8)
        def _zero(i):
            mask_v[r, pl.ds(i, LANES)] = zeros

    for cq in range(NIDS // NQ // CHUNK):
        col = q * (NIDS // NQ) + cq * CHUNK
        pltpu.sync_copy(aggr_hbm.at[pl.ds(rg * 8, 8), pl.ds(col, CHUNK)], ids_v)
        for r in range(8):
            row = jnp.full((LANES,), r, jnp.int32)

            @plsc.parallel_loop(0, CHUNK, step=LANES, unroll=16)
            def _scatter(i):
                idx = ids_v[r, pl.ds(i, LANES)]
                plsc.store_scatter(mask_v, [row, idx], ones)

    pltpu.sync_copy(mask_v, mask_hbm.at[q, pl.ds(rg * 8, 8)])


@jax.jit
def _sc_mask(aggr_info):
    mesh = plsc.VectorSubcoreMesh(core_axis_name="c", subcore_axis_name="s")
    return pl.kernel(
        _sc_mask_body,
        out_type=jax.ShapeDtypeStruct((NQ, NB, NF), jnp.float32),
        mesh=mesh,
        scratch_types=[
            pltpu.VMEM((8, CHUNK), jnp.int32),
            pltpu.VMEM((8, NF), jnp.float32),
        ],
        compiler_params=pltpu.CompilerParams(needs_layout_passes=False),
    )(aggr_info)


def _reduce_body(flow_ref, out_ref):
    for bb in range(4):
        s = flow_ref[bb, 0:128] + flow_ref[bb, 128:256]
        for half in (64, 32, 16, 8, 4, 2, 1):
            s = s[0:half] + s[half : 2 * half]
        out_ref[bb] = s[0] * jnp.float32(2.0 / NG)


@jax.jit
def _tc_reduce(flow4):
    return pl.pallas_call(
        _reduce_body,
        grid=(NB // 4,),
        in_specs=[pl.BlockSpec((4, NG // 2, 2, NF), lambda b: (b, 0, 0, 0))],
        out_specs=pl.BlockSpec((4, 2, NF), lambda b: (b, 0, 0)),
        out_shape=jax.ShapeDtypeStruct((NB, 2, NF), jnp.float32),
    )(flow4)


def _combine_body(avg_ref, prob_ref, mask_ref, out_ref):
    counts = (
        mask_ref[0] + mask_ref[1] + mask_ref[2] + mask_ref[3]
    )
    ind = jnp.where(counts > 0.0, jnp.float32(1.0), jnp.float32(0.0))
    d = (avg_ref[...] - prob_ref[...]) * ind[:, None, :]
    out_ref[0, 0] = jnp.sum(d * d) * jnp.float32(1.0 / (NB * NF * 2))


@jax.jit
def _tc_combine(avg_t, prob_t, mask):
    return pl.pallas_call(
        _combine_body,
        out_specs=pl.BlockSpec(memory_space=pltpu.SMEM),
        out_shape=jax.ShapeDtypeStruct((1, 1), jnp.float32),
    )(avg_t, prob_t, mask)


def kernel(flow, prob, aggr_info):
    flow4 = jnp.transpose(flow, (0, 2, 3, 4, 1)).reshape(NB, NG // 2, 2, NF)
    prob_t = jnp.transpose(prob, (0, 2, 1))
    mask = _sc_mask(aggr_info)
    avg_t = _tc_reduce(flow4)
    loss = _tc_combine(avg_t, prob_t, mask)[0, 0]
    avg10 = jnp.transpose(avg_t[0, :, 0:10])
    return (loss, avg10)
